# Initial kernel scaffold; baseline (speedup 1.0000x reference)
import functools

import jax
import jax.numpy as jnp
from jax import lax
from jax.experimental import pallas as pl
from jax.experimental.pallas import tpu as pltpu
from jax.experimental.pallas import tpu_sc as plsc

_PH = 7
_PW = 7
_SCALE = 7.0
_L = 16
_NC = 2
_NS = 16


def _splat(v):
    return jnp.full((_L,), v, dtype=jnp.int32)


def _prep_taps(t, size):
    valid = (t >= -1.0) & (t <= float(size))
    t0 = jnp.minimum(jnp.maximum(t, 0.0), float(size))
    tl = t0.astype(jnp.int32)
    cond = tl >= size - 1
    lo = jnp.minimum(tl, size - 1)
    hi = jnp.where(cond, size - 1, tl + 1)
    frac = jnp.where(cond, 0.0, t0 - lo.astype(jnp.float32))
    vf = jnp.where(valid, 1.0, 0.0)
    return lo, hi, (1.0 - frac) * vf, frac * vf


@functools.lru_cache(maxsize=None)
def _make_sc_kernel(N, C, H, W):
    NPTS = _PH * _PW
    OPR = C * NPTS
    NWORK = _NC * _NS
    RPW = -(-N // NWORK)
    RPW += RPW % 2
    NGRP = -(-NPTS // _L)

    mesh = plsc.VectorSubcoreMesh(core_axis_name="c", subcore_axis_name="s")

    @functools.partial(
        pl.kernel,
        out_type=jax.ShapeDtypeStruct((N, OPR), jnp.float32),
        mesh=mesh,
        scratch_types=[
            pltpu.VMEM((H * W, C), jnp.float32),
            pltpu.VMEM((5, N), jnp.float32),
            pltpu.VMEM((2, OPR), jnp.float32),
            pltpu.VMEM((4, _L), jnp.int32),
            pltpu.VMEM((4, _L), jnp.float32),
            pltpu.SemaphoreType.DMA,
            pltpu.SemaphoreType.DMA,
        ],
    )
    def sc_kernel(table_hbm, rois_hbm, out_hbm, table_v, rois_v, obuf, idx_s,
                  w_s, sem0, sem1):
        sems = (sem0, sem1)
        wid = lax.axis_index("s") * _NC + lax.axis_index("c")
        base = wid * RPW
        pltpu.sync_copy(table_hbm, table_v)
        pltpu.sync_copy(rois_hbm, rois_v)
        i16 = jnp.arange(_L, dtype=jnp.int32)
        f16 = i16.astype(jnp.float32)

        def compute_roi(groi, slot):
            gv = _splat(groi)
            sw = plsc.load_gather(rois_v, [_splat(1), gv]) * _SCALE
            sh = plsc.load_gather(rois_v, [_splat(2), gv]) * _SCALE
            ew = plsc.load_gather(rois_v, [_splat(3), gv]) * _SCALE
            eh = plsc.load_gather(rois_v, [_splat(4), gv]) * _SCALE
            bw = jnp.maximum(ew - sw, 1.0) * (1.0 / _PW)
            bh = jnp.maximum(eh - sh, 1.0) * (1.0 / _PH)
            y = sh + (f16 + 0.5) * bh
            x = sw + (f16 + 0.5) * bw
            ylo, yhi, wyl, wyh = _prep_taps(y, H)
            xlo, xhi, wxl, wxh = _prep_taps(x, W)
            idx_s[0, :] = ylo * W
            idx_s[1, :] = yhi * W
            idx_s[2, :] = xlo
            idx_s[3, :] = xhi
            w_s[0, :] = wyl
            w_s[1, :] = wyh
            w_s[2, :] = wxl
            w_s[3, :] = wxh
            rows = []
            wts = []
            for g in range(NGRP):
                p = jnp.minimum(i16 + _L * g, NPTS - 1)
                ph = p // _PW
                pw = p % _PW
                gy = [plsc.load_gather(idx_s, [_splat(t), ph]) for t in (0, 1)]
                gx = [plsc.load_gather(idx_s, [_splat(t), pw]) for t in (2, 3)]
                wy = [plsc.load_gather(w_s, [_splat(t), ph]) for t in (0, 1)]
                wx = [plsc.load_gather(w_s, [_splat(t), pw]) for t in (2, 3)]
                for a in range(2):
                    for b in range(2):
                        rows.append(gy[a] + gx[b])
                        wts.append(wy[a] * wx[b])
            slot_v = _splat(slot)
            offs = [i16 + _L * g for g in range(NGRP)]
            masks = [None] * NGRP
            if NGRP * _L > NPTS:
                masks[-1] = i16 < (NPTS - _L * (NGRP - 1))

            def cbody(c, _):
                cs = _splat(c)
                cb = _splat(c * NPTS)
                for g in range(NGRP):
                    acc = wts[4 * g] * plsc.load_gather(
                        table_v, [rows[4 * g], cs])
                    for t in range(1, 4):
                        acc = acc + wts[4 * g + t] * plsc.load_gather(
                            table_v, [rows[4 * g + t], cs])
                    plsc.store_scatter(obuf, [slot_v, cb + offs[g]], acc,
                                       mask=masks[g])
                return 0

            lax.fori_loop(0, C, cbody, 0)

        def pair_body(rr, _):
            for b in range(2):
                groi = jnp.minimum(base + 2 * rr + b, N - 1)

                @pl.when(rr > 0)
                def _wait():
                    pltpu.make_async_copy(obuf.at[b], out_hbm.at[0],
                                          sems[b]).wait()

                compute_roi(groi, b)
                pltpu.async_copy(obuf.at[b], out_hbm.at[groi], sems[b])
            return 0

        lax.fori_loop(0, RPW // 2, pair_body, 0)
        for b in range(2):
            pltpu.make_async_copy(obuf.at[b], out_hbm.at[0], sems[b]).wait()

    return sc_kernel


def kernel(features, rois):
    _, C, H, W = features.shape
    N = rois.shape[0]
    table = jnp.transpose(features[0], (1, 2, 0)).reshape(H * W, C)
    rois_t = jnp.transpose(rois)
    out = _make_sc_kernel(N, C, H, W)(table, rois_t)
    return out.reshape(N, C, _PH, _PW)


# speedup vs baseline: 1.5435x; 1.5435x over previous
import functools

import jax
import jax.numpy as jnp
from jax import lax
from jax.experimental import pallas as pl
from jax.experimental.pallas import tpu as pltpu
from jax.experimental.pallas import tpu_sc as plsc

_PH = 7
_PW = 7
_SCALE = 7.0
_L = 16
_NC = 2
_NS = 16


def _splat(v):
    return jnp.full((_L,), v, dtype=jnp.int32)


def _prep_taps(t, size):
    valid = (t >= -1.0) & (t <= float(size))
    t0 = jnp.minimum(jnp.maximum(t, 0.0), float(size))
    tl = t0.astype(jnp.int32)
    cond = tl >= size - 1
    lo = jnp.minimum(tl, size - 1)
    hi = jnp.where(cond, size - 1, tl + 1)
    frac = jnp.where(cond, 0.0, t0 - lo.astype(jnp.float32))
    vf = jnp.where(valid, 1.0, 0.0)
    return lo, hi, (1.0 - frac) * vf, frac * vf


@functools.lru_cache(maxsize=None)
def _make_sc_kernel(N, C, H, W):
    NPTS = _PH * _PW
    OPR = C * NPTS
    NWORK = _NC * _NS
    RPW = -(-N // NWORK)
    RPW += RPW % 2
    NGRP = -(-NPTS // _L)

    mesh = plsc.VectorSubcoreMesh(core_axis_name="c", subcore_axis_name="s")

    @functools.partial(
        pl.kernel,
        out_type=jax.ShapeDtypeStruct((N, OPR), jnp.float32),
        mesh=mesh,
        scratch_types=[
            pltpu.VMEM((H * W, C), jnp.float32),
            pltpu.VMEM((5, N), jnp.float32),
            pltpu.VMEM((2, OPR), jnp.float32),
            pltpu.VMEM((4, _L), jnp.int32),
            pltpu.VMEM((4, _L), jnp.float32),
            pltpu.SemaphoreType.DMA,
            pltpu.SemaphoreType.DMA,
        ],
        compiler_params=pltpu.CompilerParams(use_tc_tiling_on_sc=False,
                                             needs_layout_passes=False),
    )
    def sc_kernel(table_hbm, rois_hbm, out_hbm, table_v, rois_v, obuf, idx_s,
                  w_s, sem0, sem1):
        sems = (sem0, sem1)
        wid = lax.axis_index("s") * _NC + lax.axis_index("c")
        base = wid * RPW
        pltpu.sync_copy(table_hbm, table_v)
        pltpu.sync_copy(rois_hbm, rois_v)
        i16 = jnp.arange(_L, dtype=jnp.int32)
        f16 = i16.astype(jnp.float32)

        def compute_roi(groi, slot):
            gv = _splat(groi)
            sw = plsc.load_gather(rois_v, [_splat(1), gv]) * _SCALE
            sh = plsc.load_gather(rois_v, [_splat(2), gv]) * _SCALE
            ew = plsc.load_gather(rois_v, [_splat(3), gv]) * _SCALE
            eh = plsc.load_gather(rois_v, [_splat(4), gv]) * _SCALE
            bw = jnp.maximum(ew - sw, 1.0) * (1.0 / _PW)
            bh = jnp.maximum(eh - sh, 1.0) * (1.0 / _PH)
            y = sh + (f16 + 0.5) * bh
            x = sw + (f16 + 0.5) * bw
            ylo, yhi, wyl, wyh = _prep_taps(y, H)
            xlo, xhi, wxl, wxh = _prep_taps(x, W)
            idx_s[0, :] = ylo * W
            idx_s[1, :] = yhi * W
            idx_s[2, :] = xlo
            idx_s[3, :] = xhi
            w_s[0, :] = wyl
            w_s[1, :] = wyh
            w_s[2, :] = wxl
            w_s[3, :] = wxh
            rows = []
            wts = []
            for g in range(NGRP):
                p = jnp.minimum(i16 + _L * g, NPTS - 1)
                ph = p // _PW
                pw = p % _PW
                gy = [plsc.load_gather(idx_s, [_splat(t), ph]) for t in (0, 1)]
                gx = [plsc.load_gather(idx_s, [_splat(t), pw]) for t in (2, 3)]
                wy = [plsc.load_gather(w_s, [_splat(t), ph]) for t in (0, 1)]
                wx = [plsc.load_gather(w_s, [_splat(t), pw]) for t in (2, 3)]
                for a in range(2):
                    for b in range(2):
                        rows.append(gy[a] + gx[b])
                        wts.append(wy[a] * wx[b])
            slot_v = _splat(slot)
            offs = [i16 + _L * g for g in range(NGRP)]
            masks = [None] * NGRP
            if NGRP * _L > NPTS:
                masks[-1] = i16 < (NPTS - _L * (NGRP - 1))

            def cbody(c, _):
                cs = _splat(c)
                cb = _splat(c * NPTS)
                for g in range(NGRP):
                    acc = wts[4 * g] * plsc.load_gather(
                        table_v, [rows[4 * g], cs])
                    for t in range(1, 4):
                        acc = acc + wts[4 * g + t] * plsc.load_gather(
                            table_v, [rows[4 * g + t], cs])
                    plsc.store_scatter(obuf, [slot_v, cb + offs[g]], acc,
                                       mask=masks[g])
                return 0

            lax.fori_loop(0, C, cbody, 0)

        def pair_body(rr, _):
            for b in range(2):
                groi = jnp.minimum(base + 2 * rr + b, N - 1)

                @pl.when(rr > 0)
                def _wait():
                    pltpu.make_async_copy(obuf.at[b], out_hbm.at[0],
                                          sems[b]).wait()

                compute_roi(groi, b)
                pltpu.async_copy(obuf.at[b], out_hbm.at[groi], sems[b])
            return 0

        lax.fori_loop(0, RPW // 2, pair_body, 0)
        for b in range(2):
            pltpu.make_async_copy(obuf.at[b], out_hbm.at[0], sems[b]).wait()

    return sc_kernel


def kernel(features, rois):
    _, C, H, W = features.shape
    N = rois.shape[0]
    table = jnp.transpose(features[0], (1, 2, 0)).reshape(H * W, C)
    rois_t = jnp.transpose(rois)
    out = _make_sc_kernel(N, C, H, W)(table, rois_t)
    return out.reshape(N, C, _PH, _PW)


# speedup vs baseline: 4.5596x; 2.9541x over previous
import functools

import jax
import jax.numpy as jnp
from jax import lax
from jax.experimental import pallas as pl
from jax.experimental.pallas import tpu as pltpu
from jax.experimental.pallas import tpu_sc as plsc

_PH = 7
_PW = 7
_SCALE = 7.0
_L = 16
_NC = 2
_NS = 16


def _splat(v):
    return jnp.full((_L,), v, dtype=jnp.int32)


def _prep_taps(t, size):
    valid = (t >= -1.0) & (t <= float(size))
    t0 = jnp.minimum(jnp.maximum(t, 0.0), float(size))
    tl = t0.astype(jnp.int32)
    cond = tl >= size - 1
    lo = jnp.minimum(tl, size - 1)
    hi = jnp.where(cond, size - 1, tl + 1)
    frac = jnp.where(cond, 0.0, t0 - lo.astype(jnp.float32))
    vf = jnp.where(valid, 1.0, 0.0)
    return lo, hi, (1.0 - frac) * vf, frac * vf


@functools.lru_cache(maxsize=None)
def _make_sc_kernel(N, C, H, W):
    NPTS = _PH * _PW
    OPR = C * NPTS
    NWORK = _NC * _NS
    RPW = -(-N // NWORK)
    RPW += RPW % 2
    NGRP = -(-NPTS // _L)

    mesh = plsc.VectorSubcoreMesh(core_axis_name="c", subcore_axis_name="s")

    @functools.partial(
        pl.kernel,
        out_type=jax.ShapeDtypeStruct((N, OPR), jnp.float32),
        mesh=mesh,
        scratch_types=[
            pltpu.VMEM((H * W, C), jnp.float32),
            pltpu.VMEM((5, N), jnp.float32),
            pltpu.VMEM((2, OPR), jnp.float32),
            pltpu.VMEM((4, _L), jnp.int32),
            pltpu.VMEM((4, _L), jnp.float32),
            pltpu.VMEM((4, 4 * _L), jnp.int32),
            pltpu.VMEM((4, 4 * _L), jnp.float32),
            pltpu.SemaphoreType.DMA,
            pltpu.SemaphoreType.DMA,
        ],
        compiler_params=pltpu.CompilerParams(use_tc_tiling_on_sc=False,
                                             needs_layout_passes=False),
    )
    def sc_kernel(table_hbm, rois_hbm, out_hbm, table_v, rois_v, obuf, idx_s,
                  w_s, r2_s, w2_s, sem0, sem1):
        sems = (sem0, sem1)
        wid = lax.axis_index("s") * _NC + lax.axis_index("c")
        base = wid * RPW
        pltpu.sync_copy(table_hbm, table_v)
        pltpu.sync_copy(rois_hbm, rois_v)
        i16 = jnp.arange(_L, dtype=jnp.int32)
        f16 = i16.astype(jnp.float32)

        def compute_roi(groi, slot):
            gv = _splat(groi)
            sw = plsc.load_gather(rois_v, [_splat(1), gv]) * _SCALE
            sh = plsc.load_gather(rois_v, [_splat(2), gv]) * _SCALE
            ew = plsc.load_gather(rois_v, [_splat(3), gv]) * _SCALE
            eh = plsc.load_gather(rois_v, [_splat(4), gv]) * _SCALE
            bw = jnp.maximum(ew - sw, 1.0) * (1.0 / _PW)
            bh = jnp.maximum(eh - sh, 1.0) * (1.0 / _PH)
            y = sh + (f16 + 0.5) * bh
            x = sw + (f16 + 0.5) * bw
            ylo, yhi, wyl, wyh = _prep_taps(y, H)
            xlo, xhi, wxl, wxh = _prep_taps(x, W)
            idx_s[0, :] = ylo * W
            idx_s[1, :] = yhi * W
            idx_s[2, :] = xlo
            idx_s[3, :] = xhi
            w_s[0, :] = wyl
            w_s[1, :] = wyh
            w_s[2, :] = wxl
            w_s[3, :] = wxh
            for g in range(NGRP):
                p = jnp.minimum(i16 + _L * g, NPTS - 1)
                ph = p // _PW
                pw = p % _PW
                gy = [plsc.load_gather(idx_s, [_splat(t), ph]) for t in (0, 1)]
                gx = [plsc.load_gather(idx_s, [_splat(t), pw]) for t in (2, 3)]
                wy = [plsc.load_gather(w_s, [_splat(t), ph]) for t in (0, 1)]
                wx = [plsc.load_gather(w_s, [_splat(t), pw]) for t in (2, 3)]
                for t, (a, b) in enumerate(((0, 0), (0, 1), (1, 0), (1, 1))):
                    r2_s[t, pl.ds(_L * g, _L)] = gy[a] + gx[b]
                    w2_s[t, pl.ds(_L * g, _L)] = wy[a] * wx[b]
            slot_v = _splat(slot)
            i_npts = i16 * NPTS

            def pbody(p, _):
                pd = pl.ds(p, _L)
                r0 = r2_s[0, pd][0]
                r1 = r2_s[1, pd][0]
                r2 = r2_s[2, pd][0]
                r3 = r2_s[3, pd][0]
                w0 = jnp.full((_L,), w2_s[0, pd][0], dtype=jnp.float32)
                w1 = jnp.full((_L,), w2_s[1, pd][0], dtype=jnp.float32)
                w2 = jnp.full((_L,), w2_s[2, pd][0], dtype=jnp.float32)
                w3 = jnp.full((_L,), w2_s[3, pd][0], dtype=jnp.float32)
                sidx = i_npts + p
                for k in range(C // _L):
                    ck = pl.ds(_L * k, _L)
                    acc = (w0 * table_v[r0, ck] + w1 * table_v[r1, ck]
                           + w2 * table_v[r2, ck] + w3 * table_v[r3, ck])
                    plsc.store_scatter(obuf, [slot_v, sidx + _L * k * NPTS],
                                       acc)
                return 0

            lax.fori_loop(0, NPTS, pbody, 0)

        def pair_body(rr, _):
            for b in range(2):
                groi = jnp.minimum(base + 2 * rr + b, N - 1)

                @pl.when(rr > 0)
                def _wait():
                    pltpu.make_async_copy(obuf.at[b], out_hbm.at[0],
                                          sems[b]).wait()

                compute_roi(groi, b)
                pltpu.async_copy(obuf.at[b], out_hbm.at[groi], sems[b])
            return 0

        lax.fori_loop(0, RPW // 2, pair_body, 0)
        for b in range(2):
            pltpu.make_async_copy(obuf.at[b], out_hbm.at[0], sems[b]).wait()

    return sc_kernel


def kernel(features, rois):
    _, C, H, W = features.shape
    N = rois.shape[0]
    table = jnp.transpose(features[0], (1, 2, 0)).reshape(H * W, C)
    rois_t = jnp.transpose(rois)
    out = _make_sc_kernel(N, C, H, W)(table, rois_t)
    return out.reshape(N, C, _PH, _PW)


# speedup vs baseline: 6.6666x; 1.4621x over previous
import functools

import jax
import jax.numpy as jnp
from jax import lax
from jax.experimental import pallas as pl
from jax.experimental.pallas import tpu as pltpu
from jax.experimental.pallas import tpu_sc as plsc

_PH = 7
_PW = 7
_SCALE = 7.0
_L = 16
_NC = 2
_NS = 16


def _splat(v):
    return jnp.full((_L,), v, dtype=jnp.int32)


def _prep_taps(t, size):
    valid = (t >= -1.0) & (t <= float(size))
    t0 = jnp.minimum(jnp.maximum(t, 0.0), float(size))
    tl = t0.astype(jnp.int32)
    cond = tl >= size - 1
    lo = jnp.minimum(tl, size - 1)
    hi = jnp.where(cond, size - 1, tl + 1)
    frac = jnp.where(cond, 0.0, t0 - lo.astype(jnp.float32))
    vf = jnp.where(valid, 1.0, 0.0)
    return lo, hi, (1.0 - frac) * vf, frac * vf


@functools.lru_cache(maxsize=None)
def _make_sc_kernel(N, C, H, W):
    NPTS = _PH * _PW
    OPR = C * NPTS
    NWORK = _NC * _NS
    RPW = -(-N // NWORK)
    RPW += RPW % 2
    NGRP = -(-NPTS // _L)

    mesh = plsc.VectorSubcoreMesh(core_axis_name="c", subcore_axis_name="s")

    @functools.partial(
        pl.kernel,
        out_type=jax.ShapeDtypeStruct((N, OPR), jnp.float32),
        mesh=mesh,
        scratch_types=[
            pltpu.VMEM((H * W, C), jnp.float32),
            pltpu.VMEM((5, N), jnp.float32),
            pltpu.VMEM((2, OPR), jnp.float32),
            pltpu.VMEM((4, _L), jnp.int32),
            pltpu.VMEM((4, _L), jnp.float32),
            pltpu.VMEM((4, 4 * _L), jnp.int32),
            pltpu.VMEM((4, 4 * _L), jnp.float32),
            pltpu.SemaphoreType.DMA,
            pltpu.SemaphoreType.DMA,
        ],
        compiler_params=pltpu.CompilerParams(use_tc_tiling_on_sc=False,
                                             needs_layout_passes=False),
    )
    def sc_kernel(table_hbm, rois_hbm, out_hbm, table_v, rois_v, obuf, idx_s,
                  w_s, r2_s, w2_s, sem0, sem1):
        sems = (sem0, sem1)
        wid = lax.axis_index("s") * _NC + lax.axis_index("c")
        base = wid * RPW
        pltpu.sync_copy(table_hbm, table_v)
        pltpu.sync_copy(rois_hbm, rois_v)
        i16 = jnp.arange(_L, dtype=jnp.int32)
        f16 = i16.astype(jnp.float32)

        def compute_roi(groi, slot):
            gv = _splat(groi)
            sw = plsc.load_gather(rois_v, [_splat(1), gv]) * _SCALE
            sh = plsc.load_gather(rois_v, [_splat(2), gv]) * _SCALE
            ew = plsc.load_gather(rois_v, [_splat(3), gv]) * _SCALE
            eh = plsc.load_gather(rois_v, [_splat(4), gv]) * _SCALE
            bw = jnp.maximum(ew - sw, 1.0) * (1.0 / _PW)
            bh = jnp.maximum(eh - sh, 1.0) * (1.0 / _PH)
            y = sh + (f16 + 0.5) * bh
            x = sw + (f16 + 0.5) * bw
            ylo, yhi, wyl, wyh = _prep_taps(y, H)
            xlo, xhi, wxl, wxh = _prep_taps(x, W)
            idx_s[0, :] = ylo * W
            idx_s[1, :] = yhi * W
            idx_s[2, :] = xlo
            idx_s[3, :] = xhi
            w_s[0, :] = wyl
            w_s[1, :] = wyh
            w_s[2, :] = wxl
            w_s[3, :] = wxh
            for g in range(NGRP):
                p = jnp.minimum(i16 + _L * g, NPTS - 1)
                ph = p // _PW
                pw = p % _PW
                gy = [plsc.load_gather(idx_s, [_splat(t), ph]) for t in (0, 1)]
                gx = [plsc.load_gather(idx_s, [_splat(t), pw]) for t in (2, 3)]
                wy = [plsc.load_gather(w_s, [_splat(t), ph]) for t in (0, 1)]
                wx = [plsc.load_gather(w_s, [_splat(t), pw]) for t in (2, 3)]
                for t, (a, b) in enumerate(((0, 0), (0, 1), (1, 0), (1, 1))):
                    r2_s[t, pl.ds(_L * g, _L)] = gy[a] + gx[b]
                    w2_s[t, pl.ds(_L * g, _L)] = wy[a] * wx[b]
            slot_v = _splat(slot)
            i_npts = i16 * NPTS

            @plsc.parallel_loop(0, NPTS, unroll=2)
            def pbody(p):
                pd = pl.ds(p, _L)
                r0 = r2_s[0, pd][0]
                r1 = r2_s[1, pd][0]
                r2 = r2_s[2, pd][0]
                r3 = r2_s[3, pd][0]
                w0 = jnp.full((_L,), w2_s[0, pd][0], dtype=jnp.float32)
                w1 = jnp.full((_L,), w2_s[1, pd][0], dtype=jnp.float32)
                w2 = jnp.full((_L,), w2_s[2, pd][0], dtype=jnp.float32)
                w3 = jnp.full((_L,), w2_s[3, pd][0], dtype=jnp.float32)
                sidx = i_npts + p
                for k in range(C // _L):
                    ck = pl.ds(_L * k, _L)
                    acc = (w0 * table_v[r0, ck] + w1 * table_v[r1, ck]
                           + w2 * table_v[r2, ck] + w3 * table_v[r3, ck])
                    plsc.store_scatter(obuf, [slot_v, sidx + _L * k * NPTS],
                                       acc)

        def pair_body(rr, _):
            for b in range(2):
                groi = jnp.minimum(base + 2 * rr + b, N - 1)

                @pl.when(rr > 0)
                def _wait():
                    pltpu.make_async_copy(obuf.at[b], out_hbm.at[0],
                                          sems[b]).wait()

                compute_roi(groi, b)
                pltpu.async_copy(obuf.at[b], out_hbm.at[groi], sems[b])
            return 0

        lax.fori_loop(0, RPW // 2, pair_body, 0)
        for b in range(2):
            pltpu.make_async_copy(obuf.at[b], out_hbm.at[0], sems[b]).wait()

    return sc_kernel


def kernel(features, rois):
    _, C, H, W = features.shape
    N = rois.shape[0]
    table = jnp.transpose(features[0], (1, 2, 0)).reshape(H * W, C)
    rois_t = jnp.transpose(rois)
    out = _make_sc_kernel(N, C, H, W)(table, rois_t)
    return out.reshape(N, C, _PH, _PW)
